# baseline (speedup 1.0000x reference)
import functools

import jax
import jax.numpy as jnp
from jax import lax
from jax.experimental import pallas as pl
from jax.experimental.pallas import tpu as pltpu
from jax.experimental.pallas import tpu_sc as plsc

_B, _N, _D = 4096, 200, 64
_NC, _NS = 2, 16
_NW = _NC * _NS
_BLK = 128
_NBLK = _N


def _embed_lookup(tokens_t, wpair):
    mesh = plsc.VectorSubcoreMesh(core_axis_name="c", subcore_axis_name="s")

    @functools.partial(
        pl.kernel,
        mesh=mesh,
        compiler_params=pltpu.CompilerParams(use_tc_tiling_on_sc=False,
                                             needs_layout_passes=False),
        out_type=jax.ShapeDtypeStruct((_N, _D // 8, _B // _BLK, 8, _BLK),
                                      jnp.float32),
        scratch_types=[
            pltpu.VMEM((_N, _BLK), jnp.int32),
            pltpu.VMEM((_BLK,), jnp.int32),
            pltpu.VMEM((_BLK,), jnp.int32),
            pltpu.VMEM((_BLK, 128), jnp.float32),
            pltpu.VMEM((_BLK, 128), jnp.float32),
            pltpu.VMEM((8, 8, _BLK), jnp.float32),
            pltpu.VMEM((8, 8, _BLK), jnp.float32),
            pltpu.SemaphoreType.DMA,
            pltpu.SemaphoreType.DMA,
            pltpu.SemaphoreType.DMA,
            pltpu.SemaphoreType.DMA,
        ],
    )
    def k(idx_hbm, table_hbm, out_hbm, idx_v, pi0, pi1, st0, st1, ob0, ob1,
          g0, g1, s0, s1):
        pidx = [pi0, pi1]
        stage = [st0, st1]
        obuf = [ob0, ob1]
        gsem = [g0, g1]
        ssem = [s0, s1]
        wid = lax.axis_index("s") * _NC + lax.axis_index("c")

        pltpu.sync_copy(idx_hbm.at[:, pl.ds(wid * _BLK, _BLK)], idx_v)

        lane = lax.iota(jnp.int32, 16)
        cidx = [lane + 16 * j for j in range(8)]

        def prep_and_gather(n, p):
            for j in range(8):
                t = idx_v[n, pl.ds(j * 16, 16)]
                pidx[p][pl.ds(j * 16, 16)] = lax.shift_right_logical(t, 1)
            pltpu.async_copy(table_hbm.at[pidx[p]], stage[p], gsem[p])

        def wait_g(p):
            pltpu.make_async_copy(table_hbm.at[pidx[p]], stage[p],
                                  gsem[p]).wait()

        def start_s(n, p):
            pltpu.async_copy(obuf[p], out_hbm.at[n, :, wid], ssem[p])

        def wait_s(p):
            pltpu.make_async_copy(obuf[p], out_hbm.at[0, :, wid],
                                  ssem[p]).wait()

        def transpose(n, p):
            low = []
            for j in range(8):
                t = idx_v[n, pl.ds(j * 16, 16)]
                low.append(lax.shift_left(lax.bitwise_and(t, 1), 6))
            for dt in range(8):
                for r in range(8):
                    off = dt * 8 + r
                    for j in range(8):
                        vals = plsc.load_gather(stage[p],
                                                [cidx[j], low[j] + off])
                        obuf[p][dt, r, pl.ds(j * 16, 16)] = vals

        prep_and_gather(0, 0)
        prep_and_gather(1, 1)

        def body(kk, carry):
            for p in range(2):
                n = 2 * kk + p
                wait_g(p)

                @pl.when(kk >= 1)
                def _():
                    wait_s(p)

                transpose(n, p)
                start_s(n, p)

                @pl.when(kk <= _NBLK // 2 - 2)
                def _():
                    prep_and_gather(n + 2, p)
            return carry

        lax.fori_loop(0, _NBLK // 2, body, 0)

        for p in range(2):
            wait_s(p)

    return k(tokens_t, wpair)


def kernel(tokens, weight):
    tokens_t = jnp.swapaxes(tokens, 0, 1).astype(jnp.int32)
    wpair = lax.optimization_barrier(
        weight.reshape(weight.shape[0] // 2, 128))
    out5 = _embed_lookup(tokens_t, wpair)
    return out5.transpose(2, 4, 0, 1, 3).reshape(_B, _N, _D)


# speedup vs baseline: 1.5239x; 1.5239x over previous
import functools

import jax
import jax.numpy as jnp
from jax import lax
from jax.experimental import pallas as pl
from jax.experimental.pallas import tpu as pltpu
from jax.experimental.pallas import tpu_sc as plsc

_B, _N, _D = 4096, 200, 64
_NC, _NS = 2, 16
_NW = _NC * _NS
_BLK = 128
_NBLK = _N


def _embed_lookup(tokens_t, weight):
    mesh = plsc.VectorSubcoreMesh(core_axis_name="c", subcore_axis_name="s")

    @functools.partial(
        pl.kernel,
        mesh=mesh,
        compiler_params=pltpu.CompilerParams(use_tc_tiling_on_sc=False,
                                             needs_layout_passes=False),
        out_type=jax.ShapeDtypeStruct((_N, _D // 8, _B // _BLK, 8, _BLK),
                                      jnp.float32),
        scratch_types=[
            pltpu.VMEM((_N, _BLK), jnp.int32),
            pltpu.VMEM((_BLK, _D), jnp.float32),
            pltpu.VMEM((_BLK, _D), jnp.float32),
            pltpu.VMEM((8, 8, _BLK), jnp.float32),
            pltpu.VMEM((8, 8, _BLK), jnp.float32),
            pltpu.SemaphoreType.DMA,
            pltpu.SemaphoreType.DMA,
            pltpu.SemaphoreType.DMA,
            pltpu.SemaphoreType.DMA,
        ],
    )
    def k(idx_hbm, table_hbm, out_hbm, idx_v, st0, st1, ob0, ob1,
          g0, g1, s0, s1):
        stage = [st0, st1]
        obuf = [ob0, ob1]
        gsem = [g0, g1]
        ssem = [s0, s1]
        wid = lax.axis_index("s") * _NC + lax.axis_index("c")

        pltpu.sync_copy(idx_hbm.at[:, pl.ds(wid * _BLK, _BLK)], idx_v)

        lane = lax.iota(jnp.int32, 16)
        cidx = [lane + 16 * j for j in range(8)]

        def start_g(n, p):
            pltpu.async_copy(table_hbm.at[idx_v.at[n]], stage[p], gsem[p])

        def wait_g(p):
            pltpu.make_async_copy(table_hbm.at[idx_v.at[0]], stage[p],
                                  gsem[p]).wait()

        def start_s(n, p):
            pltpu.async_copy(obuf[p], out_hbm.at[n, :, wid], ssem[p])

        def wait_s(p):
            pltpu.make_async_copy(obuf[p], out_hbm.at[0, :, wid],
                                  ssem[p]).wait()

        def transpose(p):
            @plsc.parallel_loop(0, _D, unroll=8)
            def _(d):
                dvec = lax.broadcast(d, (16,))
                dt = d // 8
                r = lax.rem(d, 8)
                for j in range(8):
                    vals = plsc.load_gather(stage[p], [cidx[j], dvec])
                    obuf[p][dt, r, pl.ds(j * 16, 16)] = vals

        start_g(0, 0)
        start_g(1, 1)

        def body(kk, carry):
            for p in range(2):
                n = 2 * kk + p
                wait_g(p)

                @pl.when(kk >= 1)
                def _():
                    wait_s(p)

                transpose(p)
                start_s(n, p)

                @pl.when(kk <= _NBLK // 2 - 2)
                def _():
                    start_g(n + 2, p)
            return carry

        lax.fori_loop(0, _NBLK // 2, body, 0)

        for p in range(2):
            wait_s(p)

    return k(tokens_t, weight)


def kernel(tokens, weight):
    tokens_t = jnp.swapaxes(tokens, 0, 1).astype(jnp.int32)
    out5 = _embed_lookup(tokens_t, weight)
    return out5.transpose(2, 4, 0, 1, 3).reshape(_B, _N, _D)


# speedup vs baseline: 2.4238x; 1.5905x over previous
import functools

import jax
import jax.numpy as jnp
from jax import lax
from jax.experimental import pallas as pl
from jax.experimental.pallas import tpu as pltpu
from jax.experimental.pallas import tpu_sc as plsc

_B, _N, _D = 4096, 200, 64
_NC, _NS = 2, 16
_NW = _NC * _NS
_BLK = 128
_PAD = 137
_NBLK = _N


def _embed_lookup(tokens_t, weight):
    mesh = plsc.VectorSubcoreMesh(core_axis_name="c", subcore_axis_name="s")

    @functools.partial(
        pl.kernel,
        mesh=mesh,
        compiler_params=pltpu.CompilerParams(use_tc_tiling_on_sc=False,
                                             needs_layout_passes=False),
        out_type=jax.ShapeDtypeStruct((_N, _D // 8, _B // _BLK, 8, _BLK),
                                      jnp.float32),
        scratch_types=[
            pltpu.VMEM((_N, _BLK), jnp.int32),
            pltpu.VMEM((_BLK, _D), jnp.float32),
            pltpu.VMEM((_BLK, _D), jnp.float32),
            pltpu.VMEM((8, 8, _PAD), jnp.float32),
            pltpu.VMEM((8, 8, _PAD), jnp.float32),
            pltpu.SemaphoreType.DMA,
            pltpu.SemaphoreType.DMA,
            pltpu.SemaphoreType.DMA,
            pltpu.SemaphoreType.DMA,
        ],
    )
    def k(idx_hbm, table_hbm, out_hbm, idx_v, st0, st1, ob0, ob1,
          g0, g1, s0, s1):
        stage = [st0, st1]
        obuf = [ob0, ob1]
        gsem = [g0, g1]
        ssem = [s0, s1]
        wid = lax.axis_index("s") * _NC + lax.axis_index("c")

        pltpu.sync_copy(idx_hbm.at[:, pl.ds(wid * _BLK, _BLK)], idx_v)

        lane = lax.iota(jnp.int32, 16)

        def start_g(n, p):
            pltpu.async_copy(table_hbm.at[idx_v.at[n]], stage[p], gsem[p])

        def wait_g(p):
            pltpu.make_async_copy(table_hbm.at[idx_v.at[0]], stage[p],
                                  gsem[p]).wait()

        def start_s(n, p):
            pltpu.async_copy(obuf[p].at[:, :, pl.ds(0, _BLK)],
                             out_hbm.at[n, :, wid], ssem[p])

        def wait_s(p):
            pltpu.make_async_copy(obuf[p].at[:, :, pl.ds(0, _BLK)],
                                  out_hbm.at[0, :, wid], ssem[p]).wait()

        dts = [lax.shift_right_logical(lane + 16 * j, 3) for j in range(8)]
        rs = [lax.bitwise_and(lane + 16 * j, 7) for j in range(8)]

        def transpose(p):
            @plsc.parallel_loop(0, _BLK, unroll=4)
            def _(c):
                cb = lax.broadcast(c, (16,))
                for j in range(4):
                    vals = stage[p][c, pl.ds(j * 16, 16)]
                    plsc.store_scatter(obuf[p], [dts[j], rs[j], cb], vals)

        start_g(0, 0)
        start_g(1, 1)

        def body(kk, carry):
            for p in range(2):
                n = 2 * kk + p
                wait_g(p)

                @pl.when(kk >= 1)
                def _():
                    wait_s(p)

                transpose(p)
                start_s(n, p)

                @pl.when(kk <= _NBLK // 2 - 2)
                def _():
                    start_g(n + 2, p)
            return carry

        lax.fori_loop(0, _NBLK // 2, body, 0)

        for p in range(2):
            wait_s(p)

    return k(tokens_t, weight)


def kernel(tokens, weight):
    tokens_t = jnp.swapaxes(tokens, 0, 1).astype(jnp.int32)
    out5 = _embed_lookup(tokens_t, weight)
    return out5.transpose(2, 4, 0, 1, 3).reshape(_B, _N, _D)
